# Initial kernel scaffold; baseline (speedup 1.0000x reference)
import jax
import jax.numpy as jnp
from jax.experimental import pallas as pl


def kernel(edge_index, node_attr, edge_attr, Wq, bq, Wk, bk, Wv, bv, We, be, Ws, bs, W1, b1, W2, b2, g1, be1, g2, be2):
    raise NotImplementedError("write your pallas kernel here")


# speedup vs baseline: 16.9667x; 16.9667x over previous
import math

import jax
import jax.numpy as jnp
from jax import lax
from jax.experimental import pallas as pl
from jax.experimental.pallas import tpu as pltpu
from jax.experimental.pallas import tpu_sc as plsc

N = 10000
E = 320000
D = 128
H = 4
C = D // H

NPAD = 10240
NB = 1024
EB = 4000

NWORK = 32
EPT = E // NWORK
CHUNK = 40
NCHUNK = EPT // CHUNK
RPT = NPAD // 16
DPR = NPAD // 8
DRPT = DPR // 16


def _qkv_body(na, wq, bq, wk, bk, wv, bv, qo, kvo):
    x = na[...]
    qo[...] = jnp.dot(x, wq[...], preferred_element_type=jnp.float32) + bq[...]
    kvo[:, :D] = jnp.dot(x, wk[...], preferred_element_type=jnp.float32) + bk[...]
    kvo[:, D:] = jnp.dot(x, wv[...], preferred_element_type=jnp.float32) + bv[...]


def _qkv(napad, Wq, bq, Wk, bk, Wv, bv):
    w_spec = pl.BlockSpec((D, D), lambda i: (0, 0))
    b_spec = pl.BlockSpec((1, D), lambda i: (0, 0))
    r_spec = pl.BlockSpec((NB, D), lambda i: (i, 0))
    return pl.pallas_call(
        _qkv_body,
        grid=(NPAD // NB,),
        in_specs=[r_spec, w_spec, b_spec, w_spec, b_spec, w_spec, b_spec],
        out_specs=[r_spec, pl.BlockSpec((NB, 2 * D), lambda i: (i, 0))],
        out_shape=[jax.ShapeDtypeStruct((NPAD, D), jnp.float32),
                   jax.ShapeDtypeStruct((NPAD, 2 * D), jnp.float32)],
    )(napad, Wq, bq, Wk, bk, Wv, bv)


def _eproj_body(ea, we, be_, eo):
    eo[...] = jnp.dot(ea[...], we[...], preferred_element_type=jnp.float32) + be_[...]


def _eproj(edge_attr, We, be):
    return pl.pallas_call(
        _eproj_body,
        grid=(E // EB,),
        in_specs=[
            pl.BlockSpec((EB, D), lambda i: (i, 0)),
            pl.BlockSpec((D, D), lambda i: (0, 0)),
            pl.BlockSpec((1, D), lambda i: (0, 0)),
        ],
        out_specs=pl.BlockSpec((EB, D), lambda i: (i, 0)),
        out_shape=jax.ShapeDtypeStruct((E, D), jnp.float32),
    )(edge_attr, We, be)


def _sc_attn_body(idx_hbm, q_hbm, kv_hbm, e_hbm,
                  num_out, den_out,
                  idx_a, idx_b, q_rows, kv_rows, e_rows, msg_v, den_rows,
                  ex_v, num_sh, den_sh, sem_i, sem_q, sem_kv, sem_e):
    c = lax.axis_index("c")
    s = lax.axis_index("s")
    wid = s * 2 + c
    inv = 1.0 / math.sqrt(float(C))
    lane = lax.iota(jnp.int32, 16)
    zero16 = jnp.zeros((16,), jnp.float32)

    def zrow(i, _):
        for j in range(D // 16):
            msg_v[i, pl.ds(j * 16, 16)] = zero16
            den_rows[i, pl.ds(j * 16, 16)] = zero16
        return 0
    lax.fori_loop(0, CHUNK, zrow, 0)

    rbase = s * RPT
    dbase = s * DRPT

    def zcp(i, _):
        pltpu.sync_copy(msg_v, num_sh.at[pl.ds(rbase + i * CHUNK, CHUNK)])
        return 0
    lax.fori_loop(0, RPT // CHUNK, zcp, 0)

    def zcpd(i, _):
        pltpu.sync_copy(den_rows, den_sh.at[pl.ds(dbase + i * CHUNK, CHUNK)])
        return 0
    lax.fori_loop(0, DRPT // CHUNK, zcpd, 0)
    plsc.subcore_barrier()

    cbase = wid * NCHUNK

    def _do_chunk(cur, nxt, cidx):
        ci = pltpu.async_copy(idx_hbm.at[cbase + cidx + 1], nxt, sem_i)
        cq = pltpu.async_copy(q_hbm.at[cur.at[1]], q_rows, sem_q)
        ckv = pltpu.async_copy(kv_hbm.at[cur.at[0]], kv_rows, sem_kv)
        ebase = (cbase + cidx) * CHUNK
        ce = pltpu.async_copy(e_hbm.at[pl.ds(ebase, CHUNK)], e_rows, sem_e)
        cq.wait()
        ckv.wait()
        ce.wait()

        def edge(i, _):
            row = zero16
            for h in range(H):
                sl0 = pl.ds(h * C, 16)
                sl1 = pl.ds(h * C + 16, 16)
                t = (q_rows[i, sl0] * (kv_rows[i, sl0] + e_rows[i, sl0])
                     + q_rows[i, sl1] * (kv_rows[i, sl1] + e_rows[i, sl1]))
                a = jnp.sum(t) * inv
                row = jnp.where(lane == h, jnp.full((16,), a, jnp.float32), row)
            ex = jnp.exp(jnp.where(lane < H, row, -100.0))
            for h in range(H):
                av = jnp.full((16,), ex[h], jnp.float32)
                sl0 = pl.ds(h * C, 16)
                sl1 = pl.ds(h * C + 16, 16)
                msg_v[i, sl0] = (kv_rows[i, pl.ds(D + h * C, 16)]
                                 + e_rows[i, sl0]) * av
                msg_v[i, sl1] = (kv_rows[i, pl.ds(D + h * C + 16, 16)]
                                 + e_rows[i, sl1]) * av
            iv = jnp.full((16,), i, jnp.int32)
            mb = plsc.load_gather(cur, [jnp.full((16,), 3, jnp.int32), iv])
            for j in range(D // 16):
                den_rows[i, pl.ds(j * 16, 16)] = zero16
            plsc.store_scatter(den_rows, [iv, mb + lane], ex)
            return 0
        lax.fori_loop(0, CHUNK, edge, 0)

        pltpu.sync_copy(msg_v, num_sh.at[cur.at[1]], add=True)
        pltpu.sync_copy(den_rows, den_sh.at[cur.at[2]], add=True)
        ci.wait()

    pltpu.async_copy(idx_hbm.at[cbase], idx_a, sem_i).wait()

    def chunk_body(t, _):
        _do_chunk(idx_a, idx_b, 2 * t)
        _do_chunk(idx_b, idx_a, 2 * t + 1)
        return 0
    lax.fori_loop(0, NCHUNK // 2, chunk_body, 0)

    plsc.subcore_barrier()
    pltpu.sync_copy(num_sh.at[pl.ds(rbase, RPT)], num_out.at[c, pl.ds(rbase, RPT)])
    for half in range(2):
        pltpu.sync_copy(den_sh.at[pl.ds(dbase + half * CHUNK, CHUNK)], q_rows)

        def ubatch(b2, _):
            def unode(j, _2):
                ln = b2 * CHUNK + j
                pr = jnp.full((16,), ln // 8, jnp.int32)
                off = jnp.full((16,), (ln % 8) * 16, jnp.int32)
                ex_v[j, :] = plsc.load_gather(q_rows, [pr, off + lane])
                return 0
            lax.fori_loop(0, CHUNK, unode, 0)
            pltpu.sync_copy(
                ex_v,
                den_out.at[c, pl.ds(rbase + half * (RPT // 2) + b2 * CHUNK, CHUNK)])
            return 0
        lax.fori_loop(0, (RPT // 2) // CHUNK, ubatch, 0)


def _sc_attention(idx, q, kv, e):
    mesh = plsc.VectorSubcoreMesh(core_axis_name="c", subcore_axis_name="s")
    f = pl.kernel(
        _sc_attn_body,
        mesh=mesh,
        compiler_params=pltpu.CompilerParams(needs_layout_passes=False),
        out_type=(
            jax.ShapeDtypeStruct((2, NPAD, D), jnp.float32),
            jax.ShapeDtypeStruct((2, NPAD, 16), jnp.float32),
        ),
        scratch_types=[
            pltpu.VMEM((4, CHUNK), jnp.int32),
            pltpu.VMEM((4, CHUNK), jnp.int32),
            pltpu.VMEM((CHUNK, D), jnp.float32),
            pltpu.VMEM((CHUNK, 2 * D), jnp.float32),
            pltpu.VMEM((CHUNK, D), jnp.float32),
            pltpu.VMEM((CHUNK, D), jnp.float32),
            pltpu.VMEM((CHUNK, D), jnp.float32),
            pltpu.VMEM((CHUNK, 16), jnp.float32),
            pltpu.VMEM_SHARED((NPAD, D), jnp.float32),
            pltpu.VMEM_SHARED((DPR, D), jnp.float32),
            pltpu.SemaphoreType.DMA,
            pltpu.SemaphoreType.DMA,
            pltpu.SemaphoreType.DMA,
            pltpu.SemaphoreType.DMA,
        ],
    )
    return f(idx, q, kv, e)


def _final_body(np_ref, dp_ref, na_ref, ws, bs_, w1, b1_, w2, b2_,
                g1_, be1_, g2_, be2_, o_ref):
    num = np_ref[0] + np_ref[1]
    den = dp_ref[0] + dp_ref[1]
    r16 = lax.broadcasted_iota(jnp.int32, (16, D), 0)
    c16 = lax.broadcasted_iota(jnp.int32, (16, D), 1)
    expand = (r16 == c16 // C).astype(jnp.float32)
    db = jnp.dot(den, expand, preferred_element_type=jnp.float32)
    attn_out = num / (db + 1e-16)
    na = na_ref[...]
    out = attn_out + jnp.dot(na, ws[...], preferred_element_type=jnp.float32) + bs_[...]

    mu = jnp.mean(out, axis=-1, keepdims=True)
    var = jnp.mean((out - mu) ** 2, axis=-1, keepdims=True)
    x = (out - mu) / jnp.sqrt(var + 1e-5) * g1_[...] + be1_[...]
    node1 = na + x
    h = jnp.dot(node1, w1[...], preferred_element_type=jnp.float32) + b1_[...]
    h = h * jax.nn.sigmoid(h)
    h = jnp.dot(h, w2[...], preferred_element_type=jnp.float32) + b2_[...]
    mu2 = jnp.mean(h, axis=-1, keepdims=True)
    var2 = jnp.mean((h - mu2) ** 2, axis=-1, keepdims=True)
    x2 = (h - mu2) / jnp.sqrt(var2 + 1e-5) * g2_[...] + be2_[...]
    o_ref[...] = node1 + x2


def _final(num_p, den_p, napad, Ws, bs, W1, b1, W2, b2, g1, be1, g2, be2):
    w_spec = pl.BlockSpec((D, D), lambda i: (0, 0))
    b_spec = pl.BlockSpec((1, D), lambda i: (0, 0))
    r_spec = pl.BlockSpec((NB, D), lambda i: (i, 0))
    return pl.pallas_call(
        _final_body,
        grid=(NPAD // NB,),
        in_specs=[
            pl.BlockSpec((2, NB, D), lambda i: (0, i, 0)),
            pl.BlockSpec((2, NB, 16), lambda i: (0, i, 0)),
            r_spec, w_spec, b_spec, w_spec, b_spec, w_spec, b_spec,
            b_spec, b_spec, b_spec, b_spec,
        ],
        out_specs=r_spec,
        out_shape=jax.ShapeDtypeStruct((NPAD, D), jnp.float32),
    )(num_p, den_p, napad, Ws, bs, W1, b1, W2, b2, g1, be1, g2, be2)


def kernel(edge_index, node_attr, edge_attr, Wq, bq, Wk, bk, Wv, bv, We, be,
           Ws, bs, W1, b1, W2, b2, g1, be1, g2, be2):
    src = edge_index[0]
    dst = edge_index[1]
    idx = jnp.stack([src.reshape(E // CHUNK, CHUNK),
                     dst.reshape(E // CHUNK, CHUNK),
                     (dst // 8).reshape(E // CHUNK, CHUNK),
                     ((dst % 8) * 16).reshape(E // CHUNK, CHUNK)], axis=1)
    idx = jnp.concatenate([idx, jnp.zeros((1, 4, CHUNK), jnp.int32)], axis=0)
    napad = jnp.pad(node_attr, ((0, NPAD - N), (0, 0)))
    bq2, bk2, bv2, be_2, bs2, b12, b22 = (
        x.reshape(1, D) for x in (bq, bk, bv, be, bs, b1, b2))
    g12, be1_2, g22, be2_2 = (x.reshape(1, D) for x in (g1, be1, g2, be2))

    q, kv = _qkv(napad, Wq, bq2, Wk, bk2, Wv, bv2)
    e = _eproj(edge_attr, We, be_2)
    num_p, den_p = _sc_attention(idx, q, kv, e)
    out = _final(num_p, den_p, napad, Ws, bs2, W1, b12, W2, b22,
                 g12, be1_2, g22, be2_2)
    return out[:N]


# speedup vs baseline: 17.1232x; 1.0092x over previous
import math

import jax
import jax.numpy as jnp
from jax import lax
from jax.experimental import pallas as pl
from jax.experimental.pallas import tpu as pltpu
from jax.experimental.pallas import tpu_sc as plsc

N = 10000
E = 320000
D = 128
H = 4
C = D // H

NPAD = 10240
NB = 1024
EB = 4000

NWORK = 32
EPT = E // NWORK
CHUNK = 40
NCHUNK = EPT // CHUNK
RPT = NPAD // 16
DPR = NPAD // 8
DRPT = DPR // 16


def _qkv_body(na, wq, bq, wk, bk, wv, bv, qo, kvo):
    x = na[...]
    qo[...] = jnp.dot(x, wq[...], preferred_element_type=jnp.float32) + bq[...]
    kvo[:, :D] = jnp.dot(x, wk[...], preferred_element_type=jnp.float32) + bk[...]
    kvo[:, D:] = jnp.dot(x, wv[...], preferred_element_type=jnp.float32) + bv[...]


def _qkv(napad, Wq, bq, Wk, bk, Wv, bv):
    w_spec = pl.BlockSpec((D, D), lambda i: (0, 0))
    b_spec = pl.BlockSpec((1, D), lambda i: (0, 0))
    r_spec = pl.BlockSpec((NB, D), lambda i: (i, 0))
    return pl.pallas_call(
        _qkv_body,
        grid=(NPAD // NB,),
        in_specs=[r_spec, w_spec, b_spec, w_spec, b_spec, w_spec, b_spec],
        out_specs=[r_spec, pl.BlockSpec((NB, 2 * D), lambda i: (i, 0))],
        out_shape=[jax.ShapeDtypeStruct((NPAD, D), jnp.float32),
                   jax.ShapeDtypeStruct((NPAD, 2 * D), jnp.float32)],
    )(napad, Wq, bq, Wk, bk, Wv, bv)


def _eproj_body(ea, we, be_, eo):
    eo[...] = jnp.dot(ea[...], we[...], preferred_element_type=jnp.float32) + be_[...]


def _eproj(edge_attr, We, be):
    return pl.pallas_call(
        _eproj_body,
        grid=(E // EB,),
        in_specs=[
            pl.BlockSpec((EB, D), lambda i: (i, 0)),
            pl.BlockSpec((D, D), lambda i: (0, 0)),
            pl.BlockSpec((1, D), lambda i: (0, 0)),
        ],
        out_specs=pl.BlockSpec((EB, D), lambda i: (i, 0)),
        out_shape=jax.ShapeDtypeStruct((E, D), jnp.float32),
    )(edge_attr, We, be)


def _sc_attn_body(idx_hbm, q_hbm, kv_hbm, e_hbm,
                  num_out, den_out,
                  idx_a, idx_b, q_rows, kv_rows, e_rows, msg_v, den_rows,
                  ex_v, num_sh, den_sh, sem_i, sem_q, sem_kv, sem_e):
    c = lax.axis_index("c")
    s = lax.axis_index("s")
    wid = s * 2 + c
    inv = 1.0 / math.sqrt(float(C))
    lane = lax.iota(jnp.int32, 16)
    zero16 = jnp.zeros((16,), jnp.float32)

    def zrow(i, _):
        for j in range(D // 16):
            msg_v[i, pl.ds(j * 16, 16)] = zero16
            den_rows[i, pl.ds(j * 16, 16)] = zero16
        return 0
    lax.fori_loop(0, CHUNK, zrow, 0)

    rbase = s * RPT
    dbase = s * DRPT

    def zcp(i, _):
        pltpu.sync_copy(msg_v, num_sh.at[pl.ds(rbase + i * CHUNK, CHUNK)])
        return 0
    lax.fori_loop(0, RPT // CHUNK, zcp, 0)

    def zcpd(i, _):
        pltpu.sync_copy(den_rows, den_sh.at[pl.ds(dbase + i * CHUNK, CHUNK)])
        return 0
    lax.fori_loop(0, DRPT // CHUNK, zcpd, 0)
    plsc.subcore_barrier()

    cbase = wid * NCHUNK

    def _do_chunk(cur, nxt, cidx):
        ci = pltpu.async_copy(idx_hbm.at[cbase + cidx + 1], nxt, sem_i)
        cq = pltpu.async_copy(q_hbm.at[cur.at[1]], q_rows, sem_q)
        ckv = pltpu.async_copy(kv_hbm.at[cur.at[0]], kv_rows, sem_kv)
        ebase = (cbase + cidx) * CHUNK
        ce = pltpu.async_copy(e_hbm.at[pl.ds(ebase, CHUNK)], e_rows, sem_e)
        cq.wait()
        ckv.wait()
        ce.wait()

        def edge(i, _):
            row = zero16
            for h in range(H):
                sl0 = pl.ds(h * C, 16)
                sl1 = pl.ds(h * C + 16, 16)
                t = (q_rows[i, sl0] * (kv_rows[i, sl0] + e_rows[i, sl0])
                     + q_rows[i, sl1] * (kv_rows[i, sl1] + e_rows[i, sl1]))
                a = jnp.sum(t) * inv
                row = jnp.where(lane == h, jnp.full((16,), a, jnp.float32), row)
            ex = jnp.exp(jnp.where(lane < H, row, -100.0))
            for h in range(H):
                av = jnp.full((16,), ex[h], jnp.float32)
                sl0 = pl.ds(h * C, 16)
                sl1 = pl.ds(h * C + 16, 16)
                msg_v[i, sl0] = (kv_rows[i, pl.ds(D + h * C, 16)]
                                 + e_rows[i, sl0]) * av
                msg_v[i, sl1] = (kv_rows[i, pl.ds(D + h * C + 16, 16)]
                                 + e_rows[i, sl1]) * av
            iv = jnp.full((16,), i, jnp.int32)
            mb = plsc.load_gather(cur, [jnp.full((16,), 3, jnp.int32), iv])
            for j in range(D // 16):
                den_rows[i, pl.ds(j * 16, 16)] = zero16
            plsc.store_scatter(den_rows, [iv, mb + lane], ex)
            return 0
        lax.fori_loop(0, CHUNK, edge, 0, unroll=4)

        pltpu.sync_copy(msg_v, num_sh.at[cur.at[1]], add=True)
        pltpu.sync_copy(den_rows, den_sh.at[cur.at[2]], add=True)
        ci.wait()

    pltpu.async_copy(idx_hbm.at[cbase], idx_a, sem_i).wait()

    def chunk_body(t, _):
        _do_chunk(idx_a, idx_b, 2 * t)
        _do_chunk(idx_b, idx_a, 2 * t + 1)
        return 0
    lax.fori_loop(0, NCHUNK // 2, chunk_body, 0)

    plsc.subcore_barrier()
    pltpu.sync_copy(num_sh.at[pl.ds(rbase, RPT)], num_out.at[c, pl.ds(rbase, RPT)])
    for half in range(2):
        pltpu.sync_copy(den_sh.at[pl.ds(dbase + half * CHUNK, CHUNK)], q_rows)

        def ubatch(b2, _):
            def unode(j, _2):
                ln = b2 * CHUNK + j
                pr = jnp.full((16,), ln // 8, jnp.int32)
                off = jnp.full((16,), (ln % 8) * 16, jnp.int32)
                ex_v[j, :] = plsc.load_gather(q_rows, [pr, off + lane])
                return 0
            lax.fori_loop(0, CHUNK, unode, 0)
            pltpu.sync_copy(
                ex_v,
                den_out.at[c, pl.ds(rbase + half * (RPT // 2) + b2 * CHUNK, CHUNK)])
            return 0
        lax.fori_loop(0, (RPT // 2) // CHUNK, ubatch, 0)


def _sc_attention(idx, q, kv, e):
    mesh = plsc.VectorSubcoreMesh(core_axis_name="c", subcore_axis_name="s")
    f = pl.kernel(
        _sc_attn_body,
        mesh=mesh,
        compiler_params=pltpu.CompilerParams(needs_layout_passes=False),
        out_type=(
            jax.ShapeDtypeStruct((2, NPAD, D), jnp.float32),
            jax.ShapeDtypeStruct((2, NPAD, 16), jnp.float32),
        ),
        scratch_types=[
            pltpu.VMEM((4, CHUNK), jnp.int32),
            pltpu.VMEM((4, CHUNK), jnp.int32),
            pltpu.VMEM((CHUNK, D), jnp.float32),
            pltpu.VMEM((CHUNK, 2 * D), jnp.float32),
            pltpu.VMEM((CHUNK, D), jnp.float32),
            pltpu.VMEM((CHUNK, D), jnp.float32),
            pltpu.VMEM((CHUNK, D), jnp.float32),
            pltpu.VMEM((CHUNK, 16), jnp.float32),
            pltpu.VMEM_SHARED((NPAD, D), jnp.float32),
            pltpu.VMEM_SHARED((DPR, D), jnp.float32),
            pltpu.SemaphoreType.DMA,
            pltpu.SemaphoreType.DMA,
            pltpu.SemaphoreType.DMA,
            pltpu.SemaphoreType.DMA,
        ],
    )
    return f(idx, q, kv, e)


def _final_body(np_ref, dp_ref, na_ref, ws, bs_, w1, b1_, w2, b2_,
                g1_, be1_, g2_, be2_, o_ref):
    num = np_ref[0] + np_ref[1]
    den = dp_ref[0] + dp_ref[1]
    r16 = lax.broadcasted_iota(jnp.int32, (16, D), 0)
    c16 = lax.broadcasted_iota(jnp.int32, (16, D), 1)
    expand = (r16 == c16 // C).astype(jnp.float32)
    db = jnp.dot(den, expand, preferred_element_type=jnp.float32)
    attn_out = num / (db + 1e-16)
    na = na_ref[...]
    out = attn_out + jnp.dot(na, ws[...], preferred_element_type=jnp.float32) + bs_[...]

    mu = jnp.mean(out, axis=-1, keepdims=True)
    var = jnp.mean((out - mu) ** 2, axis=-1, keepdims=True)
    x = (out - mu) / jnp.sqrt(var + 1e-5) * g1_[...] + be1_[...]
    node1 = na + x
    h = jnp.dot(node1, w1[...], preferred_element_type=jnp.float32) + b1_[...]
    h = h * jax.nn.sigmoid(h)
    h = jnp.dot(h, w2[...], preferred_element_type=jnp.float32) + b2_[...]
    mu2 = jnp.mean(h, axis=-1, keepdims=True)
    var2 = jnp.mean((h - mu2) ** 2, axis=-1, keepdims=True)
    x2 = (h - mu2) / jnp.sqrt(var2 + 1e-5) * g2_[...] + be2_[...]
    o_ref[...] = node1 + x2


def _final(num_p, den_p, napad, Ws, bs, W1, b1, W2, b2, g1, be1, g2, be2):
    w_spec = pl.BlockSpec((D, D), lambda i: (0, 0))
    b_spec = pl.BlockSpec((1, D), lambda i: (0, 0))
    r_spec = pl.BlockSpec((NB, D), lambda i: (i, 0))
    return pl.pallas_call(
        _final_body,
        grid=(NPAD // NB,),
        in_specs=[
            pl.BlockSpec((2, NB, D), lambda i: (0, i, 0)),
            pl.BlockSpec((2, NB, 16), lambda i: (0, i, 0)),
            r_spec, w_spec, b_spec, w_spec, b_spec, w_spec, b_spec,
            b_spec, b_spec, b_spec, b_spec,
        ],
        out_specs=r_spec,
        out_shape=jax.ShapeDtypeStruct((NPAD, D), jnp.float32),
    )(num_p, den_p, napad, Ws, bs, W1, b1, W2, b2, g1, be1, g2, be2)


def kernel(edge_index, node_attr, edge_attr, Wq, bq, Wk, bk, Wv, bv, We, be,
           Ws, bs, W1, b1, W2, b2, g1, be1, g2, be2):
    src = edge_index[0]
    dst = edge_index[1]
    idx = jnp.stack([src.reshape(E // CHUNK, CHUNK),
                     dst.reshape(E // CHUNK, CHUNK),
                     (dst // 8).reshape(E // CHUNK, CHUNK),
                     ((dst % 8) * 16).reshape(E // CHUNK, CHUNK)], axis=1)
    idx = jnp.concatenate([idx, jnp.zeros((1, 4, CHUNK), jnp.int32)], axis=0)
    napad = jnp.pad(node_attr, ((0, NPAD - N), (0, 0)))
    bq2, bk2, bv2, be_2, bs2, b12, b22 = (
        x.reshape(1, D) for x in (bq, bk, bv, be, bs, b1, b2))
    g12, be1_2, g22, be2_2 = (x.reshape(1, D) for x in (g1, be1, g2, be2))

    q, kv = _qkv(napad, Wq, bq2, Wk, bk2, Wv, bv2)
    e = _eproj(edge_attr, We, be_2)
    num_p, den_p = _sc_attention(idx, q, kv, e)
    out = _final(num_p, den_p, napad, Ws, bs2, W1, b12, W2, b22,
                 g12, be1_2, g22, be2_2)
    return out[:N]


# speedup vs baseline: 17.7400x; 1.0360x over previous
import math

import jax
import jax.numpy as jnp
from jax import lax
from jax.experimental import pallas as pl
from jax.experimental.pallas import tpu as pltpu
from jax.experimental.pallas import tpu_sc as plsc

N = 10000
E = 320000
D = 128
H = 4
C = D // H

NPAD = 10240
NB = 1024
EB = 4000

NWORK = 32
EPT = E // NWORK
CHUNK = 40
NCHUNK = EPT // CHUNK
RPT = NPAD // 16
DW = D + 16


def _qkv_body(na, wq, bq, wk, bk, wv, bv, qo, kvo):
    x = na[...]
    qo[...] = jnp.dot(x, wq[...], preferred_element_type=jnp.float32) + bq[...]
    kvo[:, :D] = jnp.dot(x, wk[...], preferred_element_type=jnp.float32) + bk[...]
    kvo[:, D:] = jnp.dot(x, wv[...], preferred_element_type=jnp.float32) + bv[...]


def _qkv(napad, Wq, bq, Wk, bk, Wv, bv):
    w_spec = pl.BlockSpec((D, D), lambda i: (0, 0))
    b_spec = pl.BlockSpec((1, D), lambda i: (0, 0))
    r_spec = pl.BlockSpec((NB, D), lambda i: (i, 0))
    return pl.pallas_call(
        _qkv_body,
        grid=(NPAD // NB,),
        in_specs=[r_spec, w_spec, b_spec, w_spec, b_spec, w_spec, b_spec],
        out_specs=[r_spec, pl.BlockSpec((NB, 2 * D), lambda i: (i, 0))],
        out_shape=[jax.ShapeDtypeStruct((NPAD, D), jnp.float32),
                   jax.ShapeDtypeStruct((NPAD, 2 * D), jnp.float32)],
    )(napad, Wq, bq, Wk, bk, Wv, bv)


def _eproj_body(ea, we, be_, eo):
    eo[...] = jnp.dot(ea[...], we[...], preferred_element_type=jnp.float32) + be_[...]


def _eproj(edge_attr, We, be):
    return pl.pallas_call(
        _eproj_body,
        grid=(E // EB,),
        in_specs=[
            pl.BlockSpec((EB, D), lambda i: (i, 0)),
            pl.BlockSpec((D, D), lambda i: (0, 0)),
            pl.BlockSpec((1, D), lambda i: (0, 0)),
        ],
        out_specs=pl.BlockSpec((EB, D), lambda i: (i, 0)),
        out_shape=jax.ShapeDtypeStruct((E, D), jnp.float32),
    )(edge_attr, We, be)


def _sc_attn_body(idx_hbm, q_hbm, kv_hbm, e_hbm,
                  num_out,
                  idx_a, idx_b, q_rows, kv_rows, e_rows, msg_v,
                  num_sh, sem_i, sem_q, sem_kv, sem_e):
    c = lax.axis_index("c")
    s = lax.axis_index("s")
    wid = s * 2 + c
    inv = 1.0 / math.sqrt(float(C))
    lane = lax.iota(jnp.int32, 16)
    zero16 = jnp.zeros((16,), jnp.float32)

    def zrow(i, _):
        for j in range(DW // 16):
            msg_v[i, pl.ds(j * 16, 16)] = zero16
        return 0
    lax.fori_loop(0, CHUNK, zrow, 0)

    rbase = s * RPT

    def zcp(i, _):
        pltpu.sync_copy(msg_v, num_sh.at[pl.ds(rbase + i * CHUNK, CHUNK)])
        return 0
    lax.fori_loop(0, RPT // CHUNK, zcp, 0)
    plsc.subcore_barrier()

    cbase = wid * NCHUNK

    def _do_chunk(cur, nxt, cidx):
        ci = pltpu.async_copy(idx_hbm.at[cbase + cidx + 1], nxt, sem_i)
        cq = pltpu.async_copy(q_hbm.at[cur.at[1]], q_rows, sem_q)
        ckv = pltpu.async_copy(kv_hbm.at[cur.at[0]], kv_rows, sem_kv)
        ebase = (cbase + cidx) * CHUNK
        ce = pltpu.async_copy(e_hbm.at[pl.ds(ebase, CHUNK)], e_rows, sem_e)
        cq.wait()
        ckv.wait()
        ce.wait()

        def edge(i, _):
            row = zero16
            for h in range(H):
                sl0 = pl.ds(h * C, 16)
                sl1 = pl.ds(h * C + 16, 16)
                t = (q_rows[i, sl0] * (kv_rows[i, sl0] + e_rows[i, sl0])
                     + q_rows[i, sl1] * (kv_rows[i, sl1] + e_rows[i, sl1]))
                a = jnp.sum(t) * inv
                row = jnp.where(lane == h, jnp.full((16,), a, jnp.float32), row)
            ex = jnp.exp(jnp.where(lane < H, row, -100.0))
            msg_v[i, pl.ds(D, 16)] = ex
            for h in range(H):
                av = jnp.full((16,), ex[h], jnp.float32)
                sl0 = pl.ds(h * C, 16)
                sl1 = pl.ds(h * C + 16, 16)
                msg_v[i, sl0] = (kv_rows[i, pl.ds(D + h * C, 16)]
                                 + e_rows[i, sl0]) * av
                msg_v[i, sl1] = (kv_rows[i, pl.ds(D + h * C + 16, 16)]
                                 + e_rows[i, sl1]) * av
            return 0
        lax.fori_loop(0, CHUNK, edge, 0, unroll=4)

        pltpu.sync_copy(msg_v, num_sh.at[cur.at[1]], add=True)
        ci.wait()

    pltpu.async_copy(idx_hbm.at[cbase], idx_a, sem_i).wait()

    def chunk_body(t, _):
        _do_chunk(idx_a, idx_b, 2 * t)
        _do_chunk(idx_b, idx_a, 2 * t + 1)
        return 0
    lax.fori_loop(0, NCHUNK // 2, chunk_body, 0)

    plsc.subcore_barrier()
    pltpu.sync_copy(num_sh.at[pl.ds(rbase, RPT)], num_out.at[c, pl.ds(rbase, RPT)])


def _sc_attention(idx, q, kv, e):
    mesh = plsc.VectorSubcoreMesh(core_axis_name="c", subcore_axis_name="s")
    f = pl.kernel(
        _sc_attn_body,
        mesh=mesh,
        compiler_params=pltpu.CompilerParams(
            needs_layout_passes=False, use_tc_tiling_on_sc=False),
        out_type=(
            jax.ShapeDtypeStruct((2, NPAD, DW), jnp.float32),
        ),
        scratch_types=[
            pltpu.VMEM((4, CHUNK), jnp.int32),
            pltpu.VMEM((4, CHUNK), jnp.int32),
            pltpu.VMEM((CHUNK, D), jnp.float32),
            pltpu.VMEM((CHUNK, 2 * D), jnp.float32),
            pltpu.VMEM((CHUNK, D), jnp.float32),
            pltpu.VMEM((CHUNK, DW), jnp.float32),
            pltpu.VMEM_SHARED((NPAD, DW), jnp.float32),
            pltpu.SemaphoreType.DMA,
            pltpu.SemaphoreType.DMA,
            pltpu.SemaphoreType.DMA,
            pltpu.SemaphoreType.DMA,
        ],
    )
    return f(idx, q, kv, e)


def _final_body(np_ref, na_ref, ws, bs_, w1, b1_, w2, b2_,
                g1_, be1_, g2_, be2_, o_ref):
    full = np_ref[0] + np_ref[1]
    num = full[:, :D]
    den = full[:, D:]
    r16 = lax.broadcasted_iota(jnp.int32, (16, D), 0)
    c16 = lax.broadcasted_iota(jnp.int32, (16, D), 1)
    expand = (r16 == c16 // C).astype(jnp.float32)
    db = jnp.dot(den, expand, preferred_element_type=jnp.float32)
    attn_out = num / (db + 1e-16)
    na = na_ref[...]
    out = attn_out + jnp.dot(na, ws[...], preferred_element_type=jnp.float32) + bs_[...]

    mu = jnp.mean(out, axis=-1, keepdims=True)
    var = jnp.mean((out - mu) ** 2, axis=-1, keepdims=True)
    x = (out - mu) / jnp.sqrt(var + 1e-5) * g1_[...] + be1_[...]
    node1 = na + x
    h = jnp.dot(node1, w1[...], preferred_element_type=jnp.float32) + b1_[...]
    h = h * jax.nn.sigmoid(h)
    h = jnp.dot(h, w2[...], preferred_element_type=jnp.float32) + b2_[...]
    mu2 = jnp.mean(h, axis=-1, keepdims=True)
    var2 = jnp.mean((h - mu2) ** 2, axis=-1, keepdims=True)
    x2 = (h - mu2) / jnp.sqrt(var2 + 1e-5) * g2_[...] + be2_[...]
    o_ref[...] = node1 + x2


def _final(num_p, napad, Ws, bs, W1, b1, W2, b2, g1, be1, g2, be2):
    w_spec = pl.BlockSpec((D, D), lambda i: (0, 0))
    b_spec = pl.BlockSpec((1, D), lambda i: (0, 0))
    r_spec = pl.BlockSpec((NB, D), lambda i: (i, 0))
    return pl.pallas_call(
        _final_body,
        grid=(NPAD // NB,),
        in_specs=[
            pl.BlockSpec((2, NB, DW), lambda i: (0, i, 0)),
            r_spec, w_spec, b_spec, w_spec, b_spec, w_spec, b_spec,
            b_spec, b_spec, b_spec, b_spec,
        ],
        out_specs=r_spec,
        out_shape=jax.ShapeDtypeStruct((NPAD, D), jnp.float32),
    )(num_p, napad, Ws, bs, W1, b1, W2, b2, g1, be1, g2, be2)


def kernel(edge_index, node_attr, edge_attr, Wq, bq, Wk, bk, Wv, bv, We, be,
           Ws, bs, W1, b1, W2, b2, g1, be1, g2, be2):
    src = edge_index[0]
    dst = edge_index[1]
    idx = jnp.stack([src.reshape(E // CHUNK, CHUNK),
                     dst.reshape(E // CHUNK, CHUNK),
                     (dst // 8).reshape(E // CHUNK, CHUNK),
                     ((dst % 8) * 16).reshape(E // CHUNK, CHUNK)], axis=1)
    idx = jnp.concatenate([idx, jnp.zeros((1, 4, CHUNK), jnp.int32)], axis=0)
    napad = jnp.pad(node_attr, ((0, NPAD - N), (0, 0)))
    bq2, bk2, bv2, be_2, bs2, b12, b22 = (
        x.reshape(1, D) for x in (bq, bk, bv, be, bs, b1, b2))
    g12, be1_2, g22, be2_2 = (x.reshape(1, D) for x in (g1, be1, g2, be2))

    q, kv = _qkv(napad, Wq, bq2, Wk, bk2, Wv, bv2)
    e = _eproj(edge_attr, We, be_2)
    (num_p,) = _sc_attention(idx, q, kv, e)
    out = _final(num_p, napad, Ws, bs2, W1, b12, W2, b22,
                 g12, be1_2, g22, be2_2)
    return out[:N]


# speedup vs baseline: 18.3174x; 1.0325x over previous
import math

import jax
import jax.numpy as jnp
from jax import lax
from jax.experimental import pallas as pl
from jax.experimental.pallas import tpu as pltpu
from jax.experimental.pallas import tpu_sc as plsc

N = 10000
E = 320000
D = 128
H = 4
C = D // H

NPAD = 10240
NB = 1024
EB = 4000

NWORK = 32
EPT = E // NWORK
CHUNK = 40
NCHUNK = EPT // CHUNK
RPT = NPAD // 16
DW = D + 16


def _qkv_body(na, wq, bq, wk, bk, wv, bv, qo, kvo):
    x = na[...]
    qo[...] = jnp.dot(x, wq[...], preferred_element_type=jnp.float32) + bq[...]
    kvo[:, :D] = jnp.dot(x, wk[...], preferred_element_type=jnp.float32) + bk[...]
    kvo[:, D:] = jnp.dot(x, wv[...], preferred_element_type=jnp.float32) + bv[...]


def _qkv(napad, Wq, bq, Wk, bk, Wv, bv):
    w_spec = pl.BlockSpec((D, D), lambda i: (0, 0))
    b_spec = pl.BlockSpec((1, D), lambda i: (0, 0))
    r_spec = pl.BlockSpec((NB, D), lambda i: (i, 0))
    return pl.pallas_call(
        _qkv_body,
        grid=(NPAD // NB,),
        in_specs=[r_spec, w_spec, b_spec, w_spec, b_spec, w_spec, b_spec],
        out_specs=[r_spec, pl.BlockSpec((NB, 2 * D), lambda i: (i, 0))],
        out_shape=[jax.ShapeDtypeStruct((NPAD, D), jnp.float32),
                   jax.ShapeDtypeStruct((NPAD, 2 * D), jnp.float32)],
    )(napad, Wq, bq, Wk, bk, Wv, bv)


def _eproj_body(ea, we, be_, eo):
    eo[...] = jnp.dot(ea[...], we[...], preferred_element_type=jnp.float32) + be_[...]


def _eproj(edge_attr, We, be):
    return pl.pallas_call(
        _eproj_body,
        grid=(E // EB,),
        in_specs=[
            pl.BlockSpec((EB, D), lambda i: (i, 0)),
            pl.BlockSpec((D, D), lambda i: (0, 0)),
            pl.BlockSpec((1, D), lambda i: (0, 0)),
        ],
        out_specs=pl.BlockSpec((EB, D), lambda i: (i, 0)),
        out_shape=jax.ShapeDtypeStruct((E, D), jnp.float32),
    )(edge_attr, We, be)


def _sc_attn_body(idx_hbm, q_hbm, kv_hbm, e_hbm,
                  num_out,
                  idx_a, idx_b, q_rows, kv_rows, e_rows, msg_v,
                  num_sh, sem_i, sem_q, sem_kv, sem_e):
    c = lax.axis_index("c")
    s = lax.axis_index("s")
    wid = s * 2 + c
    inv = 1.0 / math.sqrt(float(C))
    lane = lax.iota(jnp.int32, 16)
    zero16 = jnp.zeros((16,), jnp.float32)

    def zrow(i, _):
        for j in range(DW // 16):
            msg_v[i, pl.ds(j * 16, 16)] = zero16
        return 0
    lax.fori_loop(0, CHUNK, zrow, 0)

    rbase = s * RPT

    def zcp(i, _):
        pltpu.sync_copy(msg_v, num_sh.at[pl.ds(rbase + i * CHUNK, CHUNK)])
        return 0
    lax.fori_loop(0, RPT // CHUNK, zcp, 0)
    plsc.subcore_barrier()

    cbase = wid * NCHUNK

    def _do_chunk(cur, nxt, cidx):
        ci = pltpu.async_copy(idx_hbm.at[cbase + cidx + 1], nxt, sem_i)
        cq = pltpu.async_copy(q_hbm.at[cur.at[1]], q_rows, sem_q)
        ckv = pltpu.async_copy(kv_hbm.at[cur.at[0]], kv_rows, sem_kv)
        ebase = (cbase + cidx) * CHUNK
        ce = pltpu.async_copy(e_hbm.at[pl.ds(ebase, CHUNK)], e_rows, sem_e)
        cq.wait()
        ckv.wait()
        ce.wait()

        def edge(i, _):
            row = zero16
            for h in range(H):
                sl0 = pl.ds(h * C, 16)
                sl1 = pl.ds(h * C + 16, 16)
                t = (q_rows[i, sl0] * (kv_rows[i, sl0] + e_rows[i, sl0])
                     + q_rows[i, sl1] * (kv_rows[i, sl1] + e_rows[i, sl1]))
                a = jnp.sum(t) * inv
                row = jnp.where(lane == h, jnp.full((16,), a, jnp.float32), row)
            ex = jnp.exp(jnp.where(lane < H, row, -100.0))
            msg_v[i, pl.ds(D, 16)] = ex
            for h in range(H):
                av = jnp.full((16,), ex[h], jnp.float32)
                sl0 = pl.ds(h * C, 16)
                sl1 = pl.ds(h * C + 16, 16)
                msg_v[i, sl0] = (kv_rows[i, pl.ds(D + h * C, 16)]
                                 + e_rows[i, sl0]) * av
                msg_v[i, sl1] = (kv_rows[i, pl.ds(D + h * C + 16, 16)]
                                 + e_rows[i, sl1]) * av
            return 0
        lax.fori_loop(0, CHUNK, edge, 0, unroll=4)

        pltpu.sync_copy(msg_v, num_sh.at[cur.at[1]], add=True)
        ci.wait()

    pltpu.async_copy(idx_hbm.at[cbase], idx_a, sem_i).wait()

    def chunk_body(t, _):
        _do_chunk(idx_a, idx_b, 2 * t)
        _do_chunk(idx_b, idx_a, 2 * t + 1)
        return 0
    lax.fori_loop(0, NCHUNK // 2, chunk_body, 0)

    plsc.subcore_barrier()
    pltpu.sync_copy(num_sh.at[pl.ds(rbase, RPT)], num_out.at[c, pl.ds(rbase, RPT)])


def _sc_attention(idx, q, kv, e):
    mesh = plsc.VectorSubcoreMesh(core_axis_name="c", subcore_axis_name="s")
    f = pl.kernel(
        _sc_attn_body,
        mesh=mesh,
        compiler_params=pltpu.CompilerParams(
            needs_layout_passes=False, use_tc_tiling_on_sc=False),
        out_type=(
            jax.ShapeDtypeStruct((2, NPAD, DW), jnp.float32),
        ),
        scratch_types=[
            pltpu.VMEM((2, CHUNK), jnp.int32),
            pltpu.VMEM((2, CHUNK), jnp.int32),
            pltpu.VMEM((CHUNK, D), jnp.float32),
            pltpu.VMEM((CHUNK, 2 * D), jnp.float32),
            pltpu.VMEM((CHUNK, D), jnp.float32),
            pltpu.VMEM((CHUNK, DW), jnp.float32),
            pltpu.VMEM_SHARED((NPAD, DW), jnp.float32),
            pltpu.SemaphoreType.DMA,
            pltpu.SemaphoreType.DMA,
            pltpu.SemaphoreType.DMA,
            pltpu.SemaphoreType.DMA,
        ],
    )
    return f(idx, q, kv, e)


def _final_body(np_ref, na_ref, ws, bs_, w1, b1_, w2, b2_,
                g1_, be1_, g2_, be2_, o_ref):
    full = np_ref[0] + np_ref[1]
    num = full[:, :D]
    den = full[:, D:]
    r16 = lax.broadcasted_iota(jnp.int32, (16, D), 0)
    c16 = lax.broadcasted_iota(jnp.int32, (16, D), 1)
    expand = (r16 == c16 // C).astype(jnp.float32)
    db = jnp.dot(den, expand, preferred_element_type=jnp.float32)
    attn_out = num / (db + 1e-16)
    na = na_ref[...]
    out = attn_out + jnp.dot(na, ws[...], preferred_element_type=jnp.float32) + bs_[...]

    mu = jnp.mean(out, axis=-1, keepdims=True)
    var = jnp.mean((out - mu) ** 2, axis=-1, keepdims=True)
    x = (out - mu) / jnp.sqrt(var + 1e-5) * g1_[...] + be1_[...]
    node1 = na + x
    h = jnp.dot(node1, w1[...], preferred_element_type=jnp.float32) + b1_[...]
    h = h * jax.nn.sigmoid(h)
    h = jnp.dot(h, w2[...], preferred_element_type=jnp.float32) + b2_[...]
    mu2 = jnp.mean(h, axis=-1, keepdims=True)
    var2 = jnp.mean((h - mu2) ** 2, axis=-1, keepdims=True)
    x2 = (h - mu2) / jnp.sqrt(var2 + 1e-5) * g2_[...] + be2_[...]
    o_ref[...] = node1 + x2


def _final(num_p, napad, Ws, bs, W1, b1, W2, b2, g1, be1, g2, be2):
    w_spec = pl.BlockSpec((D, D), lambda i: (0, 0))
    b_spec = pl.BlockSpec((1, D), lambda i: (0, 0))
    r_spec = pl.BlockSpec((NB, D), lambda i: (i, 0))
    return pl.pallas_call(
        _final_body,
        grid=(NPAD // NB,),
        in_specs=[
            pl.BlockSpec((2, NB, DW), lambda i: (0, i, 0)),
            r_spec, w_spec, b_spec, w_spec, b_spec, w_spec, b_spec,
            b_spec, b_spec, b_spec, b_spec,
        ],
        out_specs=r_spec,
        out_shape=jax.ShapeDtypeStruct((NPAD, D), jnp.float32),
    )(num_p, napad, Ws, bs, W1, b1, W2, b2, g1, be1, g2, be2)


def kernel(edge_index, node_attr, edge_attr, Wq, bq, Wk, bk, Wv, bv, We, be,
           Ws, bs, W1, b1, W2, b2, g1, be1, g2, be2):
    src = edge_index[0]
    dst = edge_index[1]
    idx = jnp.stack([src.reshape(E // CHUNK, CHUNK),
                     dst.reshape(E // CHUNK, CHUNK)], axis=1)
    idx = jnp.concatenate([idx, jnp.zeros((1, 2, CHUNK), jnp.int32)], axis=0)
    napad = jnp.pad(node_attr, ((0, NPAD - N), (0, 0)))
    bq2, bk2, bv2, be_2, bs2, b12, b22 = (
        x.reshape(1, D) for x in (bq, bk, bv, be, bs, b1, b2))
    g12, be1_2, g22, be2_2 = (x.reshape(1, D) for x in (g1, be1, g2, be2))

    q, kv = _qkv(napad, Wq, bq2, Wk, bk2, Wv, bv2)
    e = _eproj(edge_attr, We, be_2)
    (num_p,) = _sc_attention(idx, q, kv, e)
    out = _final(num_p, napad, Ws, bs2, W1, b12, W2, b22,
                 g12, be1_2, g22, be2_2)
    return out[:N]
